# initial kernel scaffold (unmeasured)
import jax
import jax.numpy as jnp
from jax import lax
from jax.experimental import pallas as pl
from jax.experimental.pallas import tpu as pltpu


def kernel(
    x,
):
    def body(*refs):
        pass

    out_shape = jax.ShapeDtypeStruct(..., jnp.float32)
    return pl.pallas_call(body, out_shape=out_shape)(...)


# baseline (device time: 404198 ns/iter reference)
import functools

import jax
import jax.numpy as jnp
from jax import lax
from jax.experimental import pallas as pl
from jax.experimental.pallas import tpu as pltpu


def kernel(x):
    m, n = x.shape
    half = m // 2

    my_y = lax.axis_index("y")
    xb_half = lax.dynamic_slice(
        x.astype(jnp.bfloat16), (my_y * half, 0), (half, n)
    )

    def body(xb_ref, out_ref, acc_ref,
             send_sem_a, recv_sem_a, send_sem_b, recv_sem_b, copy_sem):
        my_x = lax.axis_index("x")
        my_y = lax.axis_index("y")
        x_nbr = (1 - my_x, my_y)
        y_nbr = (my_x, 1 - my_y)

        barrier = pltpu.get_barrier_semaphore()
        for nbr in (x_nbr, y_nbr):
            pl.semaphore_signal(
                barrier, inc=1, device_id=nbr,
                device_id_type=pl.DeviceIdType.MESH,
            )
        pl.semaphore_wait(barrier, 2)

        rows = pl.ds(my_y * half, half)

        rdma_a = pltpu.make_async_remote_copy(
            src_ref=xb_ref,
            dst_ref=acc_ref,
            send_sem=send_sem_a,
            recv_sem=recv_sem_a,
            device_id=x_nbr,
            device_id_type=pl.DeviceIdType.MESH,
        )
        rdma_a.start()
        rdma_a.wait()

        acc_ref[...] = acc_ref[...] + xb_ref[...]

        local = pltpu.make_async_copy(acc_ref, out_ref.at[rows, :], copy_sem)
        local.start()

        rdma_b = pltpu.make_async_remote_copy(
            src_ref=acc_ref,
            dst_ref=out_ref.at[rows, :],
            send_sem=send_sem_b,
            recv_sem=recv_sem_b,
            device_id=y_nbr,
            device_id_type=pl.DeviceIdType.MESH,
        )
        rdma_b.start()
        rdma_b.wait()
        local.wait()

        @functools.partial(
            pl.run_scoped, second_barrier=pltpu.SemaphoreType.REGULAR
        )
        def _(second_barrier):
            for nbr in (x_nbr, y_nbr):
                pl.semaphore_signal(
                    second_barrier, inc=1, device_id=nbr,
                    device_id_type=pl.DeviceIdType.MESH,
                )
            pl.semaphore_wait(second_barrier, 2)

    return pl.pallas_call(
        body,
        out_shape=jax.ShapeDtypeStruct((m, n), jnp.bfloat16),
        in_specs=[pl.BlockSpec(memory_space=pltpu.VMEM)],
        out_specs=pl.BlockSpec(memory_space=pl.ANY),
        scratch_shapes=[
            pltpu.VMEM((half, n), jnp.bfloat16),
            pltpu.SemaphoreType.DMA,
            pltpu.SemaphoreType.DMA,
            pltpu.SemaphoreType.DMA,
            pltpu.SemaphoreType.DMA,
            pltpu.SemaphoreType.DMA,
        ],
        compiler_params=pltpu.CompilerParams(collective_id=0),
    )(xb_half)


# device time: 234826 ns/iter; 1.7213x vs baseline; 1.7213x over previous
import functools

import jax
import jax.numpy as jnp
from jax import lax
from jax.experimental import pallas as pl
from jax.experimental.pallas import tpu as pltpu


def kernel(x):
    m, n = x.shape
    half = m // 2

    my_y = lax.axis_index("y")
    xb_half = lax.dynamic_slice(
        x.astype(jnp.bfloat16), (my_y * half, 0), (half, n)
    )

    n_chunks = 16
    rows_per_chunk = half // n_chunks

    def body(xb_ref, out_ref, acc_ref,
             send_sems_a, recv_sems_a, send_sems_b, recv_sems_b, copy_sems):
        my_x = lax.axis_index("x")
        my_y = lax.axis_index("y")
        x_nbr = (1 - my_x, my_y)
        y_nbr = (my_x, 1 - my_y)

        barrier = pltpu.get_barrier_semaphore()
        for nbr in (x_nbr, y_nbr):
            pl.semaphore_signal(
                barrier, inc=1, device_id=nbr,
                device_id_type=pl.DeviceIdType.MESH,
            )
        pl.semaphore_wait(barrier, 2)

        row0 = my_y * half

        def chunk(k):
            return pl.ds(k * rows_per_chunk, rows_per_chunk)

        def out_chunk(k):
            return pl.ds(row0 + k * rows_per_chunk, rows_per_chunk)

        rdmas_a = []
        for k in range(n_chunks):
            rdma_a = pltpu.make_async_remote_copy(
                src_ref=xb_ref.at[chunk(k), :],
                dst_ref=acc_ref.at[chunk(k), :],
                send_sem=send_sems_a.at[k],
                recv_sem=recv_sems_a.at[k],
                device_id=x_nbr,
                device_id_type=pl.DeviceIdType.MESH,
            )
            rdma_a.start()
            rdmas_a.append(rdma_a)

        rdmas_b, locals_ = [], []
        for k in range(n_chunks):
            rdmas_a[k].wait_recv()
            acc_ref[chunk(k), :] = acc_ref[chunk(k), :] + xb_ref[chunk(k), :]
            rdma_b = pltpu.make_async_remote_copy(
                src_ref=acc_ref.at[chunk(k), :],
                dst_ref=out_ref.at[out_chunk(k), :],
                send_sem=send_sems_b.at[k],
                recv_sem=recv_sems_b.at[k],
                device_id=y_nbr,
                device_id_type=pl.DeviceIdType.MESH,
            )
            rdma_b.start()
            rdmas_b.append(rdma_b)
            local = pltpu.make_async_copy(
                acc_ref.at[chunk(k), :], out_ref.at[out_chunk(k), :],
                copy_sems.at[k],
            )
            local.start()
            locals_.append(local)

        for k in range(n_chunks):
            rdmas_b[k].wait()
            rdmas_a[k].wait_send()
            locals_[k].wait()

        @functools.partial(
            pl.run_scoped, second_barrier=pltpu.SemaphoreType.REGULAR
        )
        def _(second_barrier):
            for nbr in (x_nbr, y_nbr):
                pl.semaphore_signal(
                    second_barrier, inc=1, device_id=nbr,
                    device_id_type=pl.DeviceIdType.MESH,
                )
            pl.semaphore_wait(second_barrier, 2)

    return pl.pallas_call(
        body,
        out_shape=jax.ShapeDtypeStruct((m, n), jnp.bfloat16),
        in_specs=[pl.BlockSpec(memory_space=pltpu.VMEM)],
        out_specs=pl.BlockSpec(memory_space=pl.ANY),
        scratch_shapes=[
            pltpu.VMEM((half, n), jnp.bfloat16),
            pltpu.SemaphoreType.DMA((n_chunks,)),
            pltpu.SemaphoreType.DMA((n_chunks,)),
            pltpu.SemaphoreType.DMA((n_chunks,)),
            pltpu.SemaphoreType.DMA((n_chunks,)),
            pltpu.SemaphoreType.DMA((n_chunks,)),
        ],
        compiler_params=pltpu.CompilerParams(collective_id=0),
    )(xb_half)


# device time: 222487 ns/iter; 1.8167x vs baseline; 1.0555x over previous
import functools

import jax
import jax.numpy as jnp
from jax import lax
from jax.experimental import pallas as pl
from jax.experimental.pallas import tpu as pltpu


def kernel(x):
    m, n = x.shape
    half = m // 2
    n_chunks = 16
    rpc = half // n_chunks

    def body(x_ref, out_ref, stage_ref, send_ref, acc_ref,
             load_sems, send_sems_a, recv_sems_a,
             send_sems_b, recv_sems_b, copy_sems):
        my_x = lax.axis_index("x")
        my_y = lax.axis_index("y")
        x_nbr = (1 - my_x, my_y)
        y_nbr = (my_x, 1 - my_y)

        barrier = pltpu.get_barrier_semaphore()
        for nbr in (x_nbr, y_nbr):
            pl.semaphore_signal(
                barrier, inc=1, device_id=nbr,
                device_id_type=pl.DeviceIdType.MESH,
            )
        pl.semaphore_wait(barrier, 2)

        row0 = my_y * half

        def chunk(k):
            return pl.ds(k * rpc, rpc)

        def full_chunk(k):
            return pl.ds(row0 + k * rpc, rpc)

        def load(k):
            return pltpu.make_async_copy(
                x_ref.at[full_chunk(k), :], stage_ref.at[k % 2],
                load_sems.at[k],
            )

        load(0).start()
        load(1).start()
        rdmas_a = []
        for k in range(n_chunks):
            load(k).wait()
            send_ref[chunk(k), :] = stage_ref[k % 2].astype(jnp.bfloat16)
            rdma_a = pltpu.make_async_remote_copy(
                src_ref=send_ref.at[chunk(k), :],
                dst_ref=acc_ref.at[chunk(k), :],
                send_sem=send_sems_a.at[k],
                recv_sem=recv_sems_a.at[k],
                device_id=x_nbr,
                device_id_type=pl.DeviceIdType.MESH,
            )
            rdma_a.start()
            rdmas_a.append(rdma_a)
            if k + 2 < n_chunks:
                load(k + 2).start()

        rdmas_b, locals_ = [], []
        for k in range(n_chunks):
            rdmas_a[k].wait_recv()
            acc_ref[chunk(k), :] = acc_ref[chunk(k), :] + send_ref[chunk(k), :]
            rdma_b = pltpu.make_async_remote_copy(
                src_ref=acc_ref.at[chunk(k), :],
                dst_ref=out_ref.at[full_chunk(k), :],
                send_sem=send_sems_b.at[k],
                recv_sem=recv_sems_b.at[k],
                device_id=y_nbr,
                device_id_type=pl.DeviceIdType.MESH,
            )
            rdma_b.start()
            rdmas_b.append(rdma_b)
            local = pltpu.make_async_copy(
                acc_ref.at[chunk(k), :], out_ref.at[full_chunk(k), :],
                copy_sems.at[k],
            )
            local.start()
            locals_.append(local)

        for k in range(n_chunks):
            rdmas_b[k].wait()
            rdmas_a[k].wait_send()
            locals_[k].wait()

        @functools.partial(
            pl.run_scoped, second_barrier=pltpu.SemaphoreType.REGULAR
        )
        def _(second_barrier):
            for nbr in (x_nbr, y_nbr):
                pl.semaphore_signal(
                    second_barrier, inc=1, device_id=nbr,
                    device_id_type=pl.DeviceIdType.MESH,
                )
            pl.semaphore_wait(second_barrier, 2)

    return pl.pallas_call(
        body,
        out_shape=jax.ShapeDtypeStruct((m, n), jnp.bfloat16),
        in_specs=[pl.BlockSpec(memory_space=pl.ANY)],
        out_specs=pl.BlockSpec(memory_space=pl.ANY),
        scratch_shapes=[
            pltpu.VMEM((2, rpc, n), jnp.float32),
            pltpu.VMEM((half, n), jnp.bfloat16),
            pltpu.VMEM((half, n), jnp.bfloat16),
            pltpu.SemaphoreType.DMA((n_chunks,)),
            pltpu.SemaphoreType.DMA((n_chunks,)),
            pltpu.SemaphoreType.DMA((n_chunks,)),
            pltpu.SemaphoreType.DMA((n_chunks,)),
            pltpu.SemaphoreType.DMA((n_chunks,)),
            pltpu.SemaphoreType.DMA((n_chunks,)),
        ],
        compiler_params=pltpu.CompilerParams(
            collective_id=0, vmem_limit_bytes=60 * 1024 * 1024,
        ),
    )(x)
